# baseline (speedup 1.0000x reference)
import functools

import numpy as np
import jax
import jax.numpy as jnp
from jax import lax
from jax.experimental import pallas as pl
from jax.experimental.pallas import tpu as pltpu
from jax.experimental.pallas import tpu_sc as plsc

C_OUT = 192
PATCH = 16
K_NN = 3
_NEG = -3.0e38

_NW = 32
_GRAN = 16


def _make_sc_im2col(n_imgs, chans, gh, gw, w_pix):
    bands = n_imgs * gh
    iters = -(-bands // _NW)
    d = chans * PATCH * PATCH
    mesh = plsc.VectorSubcoreMesh(core_axis_name="c", subcore_axis_name="s")

    @functools.partial(
        pl.kernel, mesh=mesh,
        out_type=jax.ShapeDtypeStruct((bands, gw, d), jnp.float32),
        scratch_types=[
            pltpu.VMEM((chans, PATCH, w_pix), jnp.float32),
            pltpu.VMEM((gw, d), jnp.float32),
        ],
    )
    def sc_im2col(x_hbm, out_hbm, band_v, out_v):
        wid = lax.axis_index("s") * 2 + lax.axis_index("c")

        def step(t, carry):
            band = wid + t * _NW

            @pl.when(band < bands)
            def _():
                n = band // gh
                i = band % gh
                for c in range(chans):
                    pltpu.sync_copy(
                        x_hbm.at[(n * chans + c) * gh + i], band_v.at[c])

                def jloop(j, cc):
                    for c in range(chans):
                        for dh in range(PATCH):
                            vec = band_v[c, dh, pl.ds(j * PATCH, PATCH)]
                            col = (c * PATCH + dh) * PATCH
                            out_v[j, pl.ds(col, PATCH)] = vec
                    return cc

                lax.fori_loop(0, gw, jloop, 0)
                pltpu.sync_copy(out_v, out_hbm.at[band])

            return carry

        lax.fori_loop(0, iters, step, 0)

    return sc_im2col


def _feat_body(p_ref, w_ref, b_ref, o_ref):
    f = lax.dot_general(p_ref[0], w_ref[...], (((1,), (0,)), ((), ())),
                        preferred_element_type=jnp.float32)
    f = f + b_ref[...]
    n = jnp.sqrt(jnp.sum(f * f, axis=1, keepdims=True))
    o_ref[0] = f / jnp.maximum(n, 1e-12)


def _make_score_body(ways, slab, n_rows):
    inv = 1.0 / (n_rows * K_NN)

    def body(q_ref, s_ref, o_ref):
        q = q_ref[0]
        ns, l, co = s_ref.shape
        s = s_ref[...].reshape(ns * l, co)
        sim = lax.dot_general(q, s, (((1,), (1,)), ((), ())),
                              preferred_element_type=jnp.float32)
        lane = lax.broadcasted_iota(jnp.int32, (1, 128), 1)
        out = jnp.zeros((1, 128), jnp.float32)
        for c in range(ways):
            blk = sim[:, c * slab:(c + 1) * slab]
            m1 = jnp.max(blk, axis=1, keepdims=True)
            n1 = jnp.sum((blk == m1).astype(jnp.float32), axis=1,
                         keepdims=True)
            b2 = jnp.where(blk == m1, _NEG, blk)
            m2 = jnp.max(b2, axis=1, keepdims=True)
            n2 = jnp.sum((b2 == m2).astype(jnp.float32), axis=1,
                         keepdims=True)
            b3 = jnp.where(b2 == m2, _NEG, b2)
            m3 = jnp.max(b3, axis=1, keepdims=True)
            t1 = jnp.minimum(n1, float(K_NN))
            t2 = jnp.minimum(n2, jnp.maximum(float(K_NN) - t1, 0.0))
            t3 = jnp.maximum(float(K_NN) - t1 - t2, 0.0)
            s3 = m1 * t1 + m2 * t2 + m3 * t3
            tot = jnp.sum(s3) * inv
            out = out + jnp.where(lane == c, tot, 0.0)
        o_ref[...] = out[None]

    return body


def _features(patches, wmat, bias, out_index_map):
    n, l, d = patches.shape
    return pl.pallas_call(
        _feat_body,
        grid=(n,),
        in_specs=[
            pl.BlockSpec((1, l, d), lambda i: (i, 0, 0)),
            pl.BlockSpec((d, C_OUT), lambda i: (0, 0)),
            pl.BlockSpec((1, C_OUT), lambda i: (0, 0)),
        ],
        out_specs=pl.BlockSpec((1, l, C_OUT), out_index_map),
        out_shape=jax.ShapeDtypeStruct((n, l, C_OUT), jnp.float32),
    )(patches, wmat, bias)


def kernel(support_images, support_labels, query_images, Wb, bb):
    ns = support_images.shape[0]
    nq = query_images.shape[0]
    ways = support_labels.shape[1]
    per_class = ns // ways

    chans, h, w = support_images.shape[1:]
    gh, gw = h // PATCH, w // PATCH
    l = gh * gw
    d = chans * PATCH * PATCH

    im2col = _make_sc_im2col(ns, chans, gh, gw, w)
    p_s = im2col(support_images.reshape(ns * chans * gh, PATCH, w))
    p_q = im2col(query_images.reshape(nq * chans * gh, PATCH, w))

    wmat = Wb.reshape(C_OUT, d).T
    bias = bb.reshape(1, C_OUT)

    s_feats = _features(
        p_s.reshape(ns, l, d), wmat, bias,
        lambda i: ((i % ways) * per_class + i // ways, 0, 0))
    q_feats = _features(p_q.reshape(nq, l, d), wmat, bias,
                        lambda i: (i, 0, 0))

    slab = per_class * l

    scores_pad = pl.pallas_call(
        _make_score_body(ways, slab, l),
        grid=(nq,),
        in_specs=[
            pl.BlockSpec((1, l, C_OUT), lambda q: (q, 0, 0)),
            pl.BlockSpec((ns, l, C_OUT), lambda q: (0, 0, 0)),
        ],
        out_specs=pl.BlockSpec((1, 1, 128), lambda q: (q, 0, 0)),
        out_shape=jax.ShapeDtypeStruct((nq, 1, 128), jnp.float32),
    )(q_feats, s_feats)

    return scores_pad[:, 0, :ways]


# speedup vs baseline: 1.0936x; 1.0936x over previous
import functools

import numpy as np
import jax
import jax.numpy as jnp
from jax import lax
from jax.experimental import pallas as pl
from jax.experimental.pallas import tpu as pltpu
from jax.experimental.pallas import tpu_sc as plsc

C_OUT = 192
PATCH = 16
K_NN = 3
_NEG = -3.0e38

_NW = 32
_GRAN = 16


def _make_sc_im2col(n_imgs, chans, gh, gw, w_pix):
    bands = n_imgs * gh
    iters = -(-bands // _NW)
    d = chans * PATCH * PATCH
    mesh = plsc.VectorSubcoreMesh(core_axis_name="c", subcore_axis_name="s")

    row = PATCH * w_pix

    @functools.partial(
        pl.kernel, mesh=mesh,
        out_type=jax.ShapeDtypeStruct((bands, gw, d), jnp.float32),
        scratch_types=[
            pltpu.VMEM((2, chans, row), jnp.float32),
            pltpu.VMEM((2, gw, d), jnp.float32),
            pltpu.SemaphoreType.DMA,
            pltpu.SemaphoreType.DMA,
            pltpu.SemaphoreType.DMA,
            pltpu.SemaphoreType.DMA,
        ],
    )
    def sc_im2col(x_hbm, out_hbm, band_v, out_v, isem0, isem1, osem0, osem1):
        wid = lax.axis_index("s") * 2 + lax.axis_index("c")
        isems = (isem0, isem1)
        osems = (osem0, osem1)

        def band_of(t):
            band = wid + t * _NW
            return jnp.where(band < bands, band, wid)

        def in_copy(t):
            band = band_of(t)
            b = t % 2
            return pltpu.make_async_copy(
                x_hbm.at[band // gh, :, band % gh], band_v.at[b], isems[b])

        def out_copy(t):
            b = t % 2
            return pltpu.make_async_copy(
                out_v.at[b], out_hbm.at[band_of(t)], osems[b])

        in_copy(0).start()
        for t in range(iters):
            if t + 1 < iters:
                in_copy(t + 1).start()
            in_copy(t).wait()
            if t >= 2:
                out_copy(t - 2).wait()
            b = t % 2

            def jloop(j, cc, b=b):
                for c in range(chans):
                    for dh in range(PATCH):
                        vec = band_v[b, c, pl.ds(dh * w_pix + j * PATCH,
                                                 PATCH)]
                        col = (c * PATCH + dh) * PATCH
                        out_v[b, j, pl.ds(col, PATCH)] = vec
                return cc

            lax.fori_loop(0, gw, jloop, 0)
            out_copy(t).start()
        out_copy(iters - 2).wait()
        out_copy(iters - 1).wait()

    return sc_im2col


def _feat_body(p_ref, w_ref, b_ref, o_ref):
    f = lax.dot_general(p_ref[0], w_ref[...], (((1,), (0,)), ((), ())),
                        preferred_element_type=jnp.float32)
    f = f + b_ref[...]
    n = jnp.sqrt(jnp.sum(f * f, axis=1, keepdims=True))
    o_ref[0] = f / jnp.maximum(n, 1e-12)


def _make_score_body(ways, slab, n_rows):
    inv = 1.0 / (n_rows * K_NN)

    def body(q_ref, s_ref, o_ref):
        q = q_ref[0]
        ns, l, co = s_ref.shape
        s = s_ref[...].reshape(ns * l, co)
        sim = lax.dot_general(q, s, (((1,), (1,)), ((), ())),
                              preferred_element_type=jnp.float32)
        lane = lax.broadcasted_iota(jnp.int32, (1, 128), 1)
        out = jnp.zeros((1, 128), jnp.float32)
        for c in range(ways):
            blk = sim[:, c * slab:(c + 1) * slab]
            m1 = jnp.max(blk, axis=1, keepdims=True)
            n1 = jnp.sum((blk == m1).astype(jnp.float32), axis=1,
                         keepdims=True)
            b2 = jnp.where(blk == m1, _NEG, blk)
            m2 = jnp.max(b2, axis=1, keepdims=True)
            n2 = jnp.sum((b2 == m2).astype(jnp.float32), axis=1,
                         keepdims=True)
            b3 = jnp.where(b2 == m2, _NEG, b2)
            m3 = jnp.max(b3, axis=1, keepdims=True)
            t1 = jnp.minimum(n1, float(K_NN))
            t2 = jnp.minimum(n2, jnp.maximum(float(K_NN) - t1, 0.0))
            t3 = jnp.maximum(float(K_NN) - t1 - t2, 0.0)
            s3 = m1 * t1 + m2 * t2 + m3 * t3
            tot = jnp.sum(s3) * inv
            out = out + jnp.where(lane == c, tot, 0.0)
        o_ref[...] = out[None]

    return body


def _features(patches, wmat, bias, out_index_map):
    n, l, d = patches.shape
    return pl.pallas_call(
        _feat_body,
        grid=(n,),
        in_specs=[
            pl.BlockSpec((1, l, d), lambda i: (i, 0, 0)),
            pl.BlockSpec((d, C_OUT), lambda i: (0, 0)),
            pl.BlockSpec((1, C_OUT), lambda i: (0, 0)),
        ],
        out_specs=pl.BlockSpec((1, l, C_OUT), out_index_map),
        out_shape=jax.ShapeDtypeStruct((n, l, C_OUT), jnp.float32),
    )(patches, wmat, bias)


def kernel(support_images, support_labels, query_images, Wb, bb):
    ns = support_images.shape[0]
    nq = query_images.shape[0]
    ways = support_labels.shape[1]
    per_class = ns // ways

    chans, h, w = support_images.shape[1:]
    gh, gw = h // PATCH, w // PATCH
    l = gh * gw
    d = chans * PATCH * PATCH

    im2col = _make_sc_im2col(ns, chans, gh, gw, w)
    p_s = im2col(support_images.reshape(ns, chans, gh, PATCH * w))
    p_q = im2col(query_images.reshape(nq, chans, gh, PATCH * w))

    wmat = Wb.reshape(C_OUT, d).T
    bias = bb.reshape(1, C_OUT)

    s_feats = _features(
        p_s.reshape(ns, l, d), wmat, bias,
        lambda i: ((i % ways) * per_class + i // ways, 0, 0))
    q_feats = _features(p_q.reshape(nq, l, d), wmat, bias,
                        lambda i: (i, 0, 0))

    slab = per_class * l

    scores_pad = pl.pallas_call(
        _make_score_body(ways, slab, l),
        grid=(nq,),
        in_specs=[
            pl.BlockSpec((1, l, C_OUT), lambda q: (q, 0, 0)),
            pl.BlockSpec((ns, l, C_OUT), lambda q: (0, 0, 0)),
        ],
        out_specs=pl.BlockSpec((1, 1, 128), lambda q: (q, 0, 0)),
        out_shape=jax.ShapeDtypeStruct((nq, 1, 128), jnp.float32),
    )(q_feats, s_feats)

    return scores_pad[:, 0, :ways]
